# Initial kernel scaffold; baseline (speedup 1.0000x reference)
import jax
import jax.numpy as jnp
from jax.experimental import pallas as pl


def kernel(x, edge_index, W1, att_src1, att_dst1, b1, W2, att_src2, att_dst2, b2):
    raise NotImplementedError("write your pallas kernel here")


# speedup vs baseline: 5.2093x; 5.2093x over previous
import functools

import jax
import jax.numpy as jnp
from jax.experimental import pallas as pl


NEG_SLOPE = 0.2


def _mm_body(x_ref, w_ref, o_ref):
    o_ref[...] = jnp.dot(x_ref[...], w_ref[...],
                         preferred_element_type=jnp.float32)


def _matmul(x, w, bn=400):
    n, k = x.shape
    m = w.shape[1]
    return pl.pallas_call(
        _mm_body,
        grid=(n // bn,),
        in_specs=[pl.BlockSpec((bn, k), lambda i: (i, 0)),
                  pl.BlockSpec((k, m), lambda i: (0, 0))],
        out_specs=pl.BlockSpec((bn, m), lambda i: (i, 0)),
        out_shape=jax.ShapeDtypeStruct((n, m), jnp.float32),
    )(x, w)


def _leaky(a):
    return jnp.where(a > 0, a, NEG_SLOPE * a)


def _gat_layer(h, src, dst, att_src, att_dst, heads, ch):
    n = h.shape[0]
    hh = h.reshape(n, heads, ch)
    s_src = (hh * att_src).sum(-1)
    s_dst = (hh * att_dst).sum(-1)
    c = s_src.max(0) + s_dst.max(0)

    p = jnp.exp(_leaky(s_src[src] + s_dst[dst]) - c)
    msg = h[src] * jnp.repeat(p, ch, axis=1)
    acc = jax.ops.segment_sum(jnp.concatenate([msg, p], 1), dst,
                              num_segments=n)

    p_self = jnp.exp(_leaky(s_src + s_dst) - c)
    acc = acc + jnp.concatenate(
        [h * jnp.repeat(p_self, ch, axis=1), p_self], 1)

    num = acc[:, :heads * ch]
    den = jnp.repeat(acc[:, heads * ch:], ch, axis=1) + 1e-16
    return num / den


def kernel(x, edge_index, W1, att_src1, att_dst1, b1,
           W2, att_src2, att_dst2, b2):
    src = edge_index[0].astype(jnp.int32)
    dst = edge_index[1].astype(jnp.int32)

    h1 = _matmul(x, W1)
    o1 = _gat_layer(h1, src, dst, att_src1, att_dst1, 8, 16) + b1
    h1a = jax.nn.elu(o1)

    h2 = _matmul(h1a, W2)
    o2 = _gat_layer(h2, src, dst, att_src2, att_dst2, 1, 32) + b2
    return jax.nn.log_softmax(o2, axis=1)


# speedup vs baseline: 58.9695x; 11.3200x over previous
import functools

import jax
import jax.numpy as jnp
from jax import lax
from jax.experimental import pallas as pl
from jax.experimental.pallas import tpu as pltpu
from jax.experimental.pallas import tpu_sc as plsc

F32 = jnp.float32
NEG_SLOPE = 0.2
N = 10000
E = 320000
NC, NS, LANES = 2, 16, 16
NW = NC * NS
EPW = E // NW
CHUNK = 80
NCHUNK = EPW // CHUNK
RPS = N // NS
BN = 400
GRID = N // BN


def _leaky(a):
    return jnp.where(a > 0, a, NEG_SLOPE * a)


def _pre1_body(x_ref, w_ref, a1_ref, table_ref, ssd_ref, sdst_ref, mx_ref):
    h = jnp.dot(x_ref[...], w_ref[...], preferred_element_type=F32)
    ssd = jnp.dot(h, a1_ref[...], preferred_element_type=F32)
    table_ref[...] = jnp.concatenate(
        [h, ssd[:, 0:8], jnp.zeros((BN, 8), F32)], axis=1)
    ssd_ref[...] = ssd
    sdst_ref[...] = jnp.concatenate(
        [ssd[:, 8:16], jnp.zeros((BN, 8), F32)], axis=1)
    mx_ref[...] = jnp.broadcast_to(jnp.max(ssd, axis=0, keepdims=True),
                                   (8, 16))


def _pre1(x, W1, A1):
    return pl.pallas_call(
        _pre1_body,
        grid=(GRID,),
        in_specs=[pl.BlockSpec((BN, 128), lambda i: (i, 0)),
                  pl.BlockSpec((128, 128), lambda i: (0, 0)),
                  pl.BlockSpec((128, 16), lambda i: (0, 0))],
        out_specs=[pl.BlockSpec((BN, 144), lambda i: (i, 0)),
                   pl.BlockSpec((BN, 16), lambda i: (i, 0)),
                   pl.BlockSpec((BN, 16), lambda i: (i, 0)),
                   pl.BlockSpec((8, 16), lambda i: (i, 0))],
        out_shape=[jax.ShapeDtypeStruct((N, 144), F32),
                   jax.ShapeDtypeStruct((N, 16), F32),
                   jax.ShapeDtypeStruct((N, 16), F32),
                   jax.ShapeDtypeStruct((8 * GRID, 16), F32)],
    )(x, W1, A1)


def _mid_body(acc0_ref, acc1_ref, table_ref, ssd_ref, c1_ref, b1_ref, w2_ref,
              a2_ref, r1_ref, table2_ref, sdst2_ref, mx2_ref):
    h1 = table_ref[:, 0:128]
    ssd = ssd_ref[...]
    p_self = jnp.exp(_leaky(ssd[:, 0:8] + ssd[:, 8:16]) - c1_ref[0:1, 0:8])
    acc = acc0_ref[...] + acc1_ref[...]
    num = acc[:, 0:128] + h1 * jnp.dot(p_self, r1_ref[...],
                                       preferred_element_type=F32)
    den = jnp.dot(acc[:, 128:136] + p_self, r1_ref[...],
                  preferred_element_type=F32) + 1e-16
    o1 = num / den + b1_ref[...]
    h1a = jnp.where(o1 > 0, o1, jnp.exp(jnp.minimum(o1, 0.0)) - 1.0)
    h2 = jnp.dot(h1a, w2_ref[...], preferred_element_type=F32)
    ssd2 = jnp.dot(h2, a2_ref[...], preferred_element_type=F32)
    table2_ref[...] = jnp.concatenate([h2, ssd2], axis=1)
    sdst2_ref[...] = jnp.concatenate(
        [ssd2[:, 1:2], jnp.zeros((BN, 15), F32)], axis=1)
    mx2_ref[...] = jnp.broadcast_to(jnp.max(ssd2, axis=0, keepdims=True),
                                    (8, 16))


def _mid(acc1_pair, table1, ssd1, c1b, b1row, W2, A2, R1):
    return pl.pallas_call(
        _mid_body,
        grid=(GRID,),
        in_specs=[pl.BlockSpec((BN, 144), lambda i: (i, 0)),
                  pl.BlockSpec((BN, 144), lambda i: (i + GRID, 0)),
                  pl.BlockSpec((BN, 144), lambda i: (i, 0)),
                  pl.BlockSpec((BN, 16), lambda i: (i, 0)),
                  pl.BlockSpec((8, 16), lambda i: (0, 0)),
                  pl.BlockSpec((1, 128), lambda i: (0, 0)),
                  pl.BlockSpec((128, 32), lambda i: (0, 0)),
                  pl.BlockSpec((32, 16), lambda i: (0, 0)),
                  pl.BlockSpec((8, 128), lambda i: (0, 0))],
        out_specs=[pl.BlockSpec((BN, 48), lambda i: (i, 0)),
                   pl.BlockSpec((BN, 16), lambda i: (i, 0)),
                   pl.BlockSpec((8, 16), lambda i: (i, 0))],
        out_shape=[jax.ShapeDtypeStruct((N, 48), F32),
                   jax.ShapeDtypeStruct((N, 16), F32),
                   jax.ShapeDtypeStruct((8 * GRID, 16), F32)],
    )(acc1_pair, acc1_pair, table1, ssd1, c1b, b1row, W2, A2, R1)


def _out_body(acc0_ref, acc1_ref, table2_ref, c2_ref, b2_ref, o_ref):
    h2 = table2_ref[:, 0:32]
    s2s = table2_ref[:, 32:33]
    s2d = table2_ref[:, 33:34]
    p_self = jnp.exp(_leaky(s2s + s2d) - c2_ref[0:1, 0:1])
    acc = acc0_ref[...] + acc1_ref[...]
    num = acc[:, 0:32] + h2 * p_self
    den = acc[:, 32:33] + p_self + 1e-16
    o2 = num / den + b2_ref[...]
    m = jnp.max(o2, axis=1, keepdims=True)
    z = o2 - m
    o_ref[...] = z - jnp.log(jnp.sum(jnp.exp(z), axis=1, keepdims=True))


def _out(acc2_pair, table2, c2b, b2row):
    return pl.pallas_call(
        _out_body,
        grid=(GRID,),
        in_specs=[pl.BlockSpec((BN, 48), lambda i: (i, 0)),
                  pl.BlockSpec((BN, 48), lambda i: (i + GRID, 0)),
                  pl.BlockSpec((BN, 48), lambda i: (i, 0)),
                  pl.BlockSpec((8, 16), lambda i: (0, 0)),
                  pl.BlockSpec((1, 32), lambda i: (0, 0))],
        out_specs=pl.BlockSpec((BN, 32), lambda i: (i, 0)),
        out_shape=jax.ShapeDtypeStruct((N, 32), F32),
    )(acc2_pair, acc2_pair, table2, c2b, b2row)


def _make_sc_edge(DT, H, CH):
    p_col = H * CH
    h_vecs = p_col // 16
    mesh = plsc.VectorSubcoreMesh(core_axis_name="c", subcore_axis_name="s")

    @functools.partial(
        pl.kernel,
        mesh=mesh,
        compiler_params=pltpu.CompilerParams(use_tc_tiling_on_sc=False,
                                             needs_layout_passes=False),
        out_type=jax.ShapeDtypeStruct((NC * N, DT), F32),
        scratch_types=[
            pltpu.VMEM_SHARED((N, DT), F32),
            pltpu.VMEM((NCHUNK, CHUNK), jnp.int32),
            pltpu.VMEM((NCHUNK, CHUNK), jnp.int32),
            pltpu.VMEM((CHUNK, DT), F32),
            pltpu.VMEM((CHUNK, 16), F32),
            pltpu.VMEM((16,), F32),
            pltpu.SemaphoreType.DMA,
            pltpu.SemaphoreType.DMA,
        ],
    )
    def sc_edge(table_hbm, sdst_hbm, c_hbm, src_hbm, dst_hbm, zeros_hbm,
                out_hbm, acc_sh, src_v, dst_v, rows_v, sdr_v, c_v,
                gsem, dsem):
        cid = lax.axis_index("c")
        sid = lax.axis_index("s")
        wid = sid * NC + cid
        pltpu.sync_copy(src_hbm.at[wid], src_v)
        pltpu.sync_copy(dst_hbm.at[wid], dst_v)
        pltpu.sync_copy(c_hbm, c_v)
        pltpu.sync_copy(zeros_hbm.at[pl.ds(sid * RPS, RPS)],
                        acc_sh.at[pl.ds(sid * RPS, RPS)])
        plsc.subcore_barrier()

        cvec = c_v[...]
        lane = lax.iota(jnp.int32, 16)

        def edge_body(e, carry):
            sv = rows_v[e, pl.ds(p_col, 16)]
            dv = sdr_v[e, pl.ds(0, 16)]
            p = jnp.exp(_leaky(sv + dv) - cvec)
            p = jnp.where(lane < H, p, 0.0)
            rows_v[e, pl.ds(p_col, 16)] = p
            for v in range(h_vecs):
                ps = jnp.broadcast_to(p[(v * 16) // CH], (16,))
                rows_v[e, pl.ds(v * 16, 16)] = (
                    rows_v[e, pl.ds(v * 16, 16)] * ps)
            return carry

        def chunk_body(j, carry):
            cg = pltpu.async_copy(table_hbm.at[src_v.at[j]], rows_v, gsem)
            cd = pltpu.async_copy(sdst_hbm.at[dst_v.at[j]], sdr_v, dsem)
            cg.wait()
            cd.wait()
            lax.fori_loop(0, CHUNK, edge_body, 0)
            pltpu.sync_copy(rows_v, acc_sh.at[dst_v.at[j]], add=True)
            return carry

        lax.fori_loop(0, NCHUNK, chunk_body, 0)
        plsc.subcore_barrier()
        pltpu.sync_copy(acc_sh.at[pl.ds(sid * RPS, RPS)],
                        out_hbm.at[pl.ds(cid * N + sid * RPS, RPS)])

    return sc_edge


_sc_edge1 = _make_sc_edge(144, 8, 16)
_sc_edge2 = _make_sc_edge(48, 1, 32)


def kernel(x, edge_index, W1, att_src1, att_dst1, b1,
           W2, att_src2, att_dst2, b2):
    src = edge_index[0].astype(jnp.int32).reshape(NW, NCHUNK, CHUNK)
    dst = edge_index[1].astype(jnp.int32).reshape(NW, NCHUNK, CHUNK)

    heads = jnp.arange(8, dtype=jnp.int32)
    cols = jnp.arange(128, dtype=jnp.int32)
    mask1 = (cols[:, None] // 16 == heads[None, :]).astype(F32)
    a1s = att_src1.reshape(-1)
    a1d = att_dst1.reshape(-1)
    A1 = jnp.concatenate([a1s[:, None] * mask1, a1d[:, None] * mask1], 1)
    R1 = mask1.T
    a2s = att_src2.reshape(-1)
    a2d = att_dst2.reshape(-1)
    A2 = jnp.concatenate([a2s[:, None], a2d[:, None],
                          jnp.zeros((32, 14), F32)], 1)

    zeros144 = jnp.zeros((N, 144), F32)
    zeros48 = jnp.zeros((N, 48), F32)

    table1, ssd1, sdst1, mx1 = _pre1(x, W1, A1)
    m1 = jnp.max(mx1, axis=0)
    c1 = jnp.concatenate([m1[0:8] + m1[8:16], jnp.zeros((8,), F32)])
    acc1 = _sc_edge1(table1, sdst1, c1, src, dst, zeros144)
    c1b = jnp.broadcast_to(c1[None, :], (8, 16))

    table2, sdst2, mx2 = _mid(acc1, table1, ssd1, c1b, b1.reshape(1, 128),
                              W2, A2, R1)
    m2 = jnp.max(mx2, axis=0)
    c2 = jnp.concatenate([m2[0:1] + m2[1:2], jnp.zeros((15,), F32)])
    acc2 = _sc_edge2(table2, sdst2, c2, src, dst, zeros48)
    c2b = jnp.broadcast_to(c2[None, :], (8, 16))

    return _out(acc2, table2, c2b, b2.reshape(1, 32))


# speedup vs baseline: 79.1985x; 1.3430x over previous
import functools

import jax
import jax.numpy as jnp
from jax import lax
from jax.experimental import pallas as pl
from jax.experimental.pallas import tpu as pltpu
from jax.experimental.pallas import tpu_sc as plsc

F32 = jnp.float32
NEG_SLOPE = 0.2
N = 10000
E = 320000
NC, NS, LANES = 2, 16, 16
NW = NC * NS
EPW = E // NW
CHUNK = 80
NCHUNK = EPW // CHUNK
RPS = N // NS
BN = 400
GRID = N // BN


def _leaky(a):
    return jnp.where(a > 0, a, NEG_SLOPE * a)


def _pre1_body(x_ref, w_ref, a1_ref, table_ref, ssd_ref, sdst_ref, mx_ref):
    h = jnp.dot(x_ref[...], w_ref[...], preferred_element_type=F32)
    ssd = jnp.dot(h, a1_ref[...], preferred_element_type=F32)
    table_ref[...] = jnp.concatenate(
        [h, ssd[:, 0:8], jnp.zeros((BN, 8), F32)], axis=1)
    ssd_ref[...] = ssd
    sdst_ref[...] = jnp.concatenate(
        [ssd[:, 8:16], jnp.zeros((BN, 8), F32)], axis=1)
    mx_ref[...] = jnp.broadcast_to(jnp.max(ssd, axis=0, keepdims=True),
                                   (8, 16))


def _pre1(x, W1, A1):
    return pl.pallas_call(
        _pre1_body,
        grid=(GRID,),
        in_specs=[pl.BlockSpec((BN, 128), lambda i: (i, 0)),
                  pl.BlockSpec((128, 128), lambda i: (0, 0)),
                  pl.BlockSpec((128, 16), lambda i: (0, 0))],
        out_specs=[pl.BlockSpec((BN, 144), lambda i: (i, 0)),
                   pl.BlockSpec((BN, 16), lambda i: (i, 0)),
                   pl.BlockSpec((BN, 16), lambda i: (i, 0)),
                   pl.BlockSpec((8, 16), lambda i: (i, 0))],
        out_shape=[jax.ShapeDtypeStruct((N, 144), F32),
                   jax.ShapeDtypeStruct((N, 16), F32),
                   jax.ShapeDtypeStruct((N, 16), F32),
                   jax.ShapeDtypeStruct((8 * GRID, 16), F32)],
    )(x, W1, A1)


def _mid_body(acc0_ref, acc1_ref, table_ref, ssd_ref, c1_ref, b1_ref, w2_ref,
              a2_ref, r1_ref, table2_ref, sdst2_ref, mx2_ref):
    h1 = table_ref[:, 0:128]
    ssd = ssd_ref[...]
    p_self = jnp.exp(_leaky(ssd[:, 0:8] + ssd[:, 8:16]) - c1_ref[0:1, 0:8])
    acc = acc0_ref[...] + acc1_ref[...]
    num = acc[:, 0:128] + h1 * jnp.dot(p_self, r1_ref[...],
                                       preferred_element_type=F32)
    den = jnp.dot(acc[:, 128:136] + p_self, r1_ref[...],
                  preferred_element_type=F32) + 1e-16
    o1 = num / den + b1_ref[...]
    h1a = jnp.where(o1 > 0, o1, jnp.exp(jnp.minimum(o1, 0.0)) - 1.0)
    h2 = jnp.dot(h1a, w2_ref[...], preferred_element_type=F32)
    ssd2 = jnp.dot(h2, a2_ref[...], preferred_element_type=F32)
    table2_ref[...] = jnp.concatenate([h2, ssd2], axis=1)
    sdst2_ref[...] = jnp.concatenate(
        [ssd2[:, 1:2], jnp.zeros((BN, 15), F32)], axis=1)
    mx2_ref[...] = jnp.broadcast_to(jnp.max(ssd2, axis=0, keepdims=True),
                                    (8, 16))


def _mid(acc1_pair, table1, ssd1, c1b, b1row, W2, A2, R1):
    return pl.pallas_call(
        _mid_body,
        grid=(GRID,),
        in_specs=[pl.BlockSpec((BN, 144), lambda i: (i, 0)),
                  pl.BlockSpec((BN, 144), lambda i: (i + GRID, 0)),
                  pl.BlockSpec((BN, 144), lambda i: (i, 0)),
                  pl.BlockSpec((BN, 16), lambda i: (i, 0)),
                  pl.BlockSpec((8, 16), lambda i: (0, 0)),
                  pl.BlockSpec((1, 128), lambda i: (0, 0)),
                  pl.BlockSpec((128, 32), lambda i: (0, 0)),
                  pl.BlockSpec((32, 16), lambda i: (0, 0)),
                  pl.BlockSpec((8, 128), lambda i: (0, 0))],
        out_specs=[pl.BlockSpec((BN, 48), lambda i: (i, 0)),
                   pl.BlockSpec((BN, 16), lambda i: (i, 0)),
                   pl.BlockSpec((8, 16), lambda i: (i, 0))],
        out_shape=[jax.ShapeDtypeStruct((N, 48), F32),
                   jax.ShapeDtypeStruct((N, 16), F32),
                   jax.ShapeDtypeStruct((8 * GRID, 16), F32)],
    )(acc1_pair, acc1_pair, table1, ssd1, c1b, b1row, W2, A2, R1)


def _out_body(acc0_ref, acc1_ref, table2_ref, c2_ref, b2_ref, o_ref):
    h2 = table2_ref[:, 0:32]
    s2s = table2_ref[:, 32:33]
    s2d = table2_ref[:, 33:34]
    p_self = jnp.exp(_leaky(s2s + s2d) - c2_ref[0:1, 0:1])
    acc = acc0_ref[...] + acc1_ref[...]
    num = acc[:, 0:32] + h2 * p_self
    den = acc[:, 32:33] + p_self + 1e-16
    o2 = num / den + b2_ref[...]
    m = jnp.max(o2, axis=1, keepdims=True)
    z = o2 - m
    o_ref[...] = z - jnp.log(jnp.sum(jnp.exp(z), axis=1, keepdims=True))


def _out(acc2_pair, table2, c2b, b2row):
    return pl.pallas_call(
        _out_body,
        grid=(GRID,),
        in_specs=[pl.BlockSpec((BN, 48), lambda i: (i, 0)),
                  pl.BlockSpec((BN, 48), lambda i: (i + GRID, 0)),
                  pl.BlockSpec((BN, 48), lambda i: (i, 0)),
                  pl.BlockSpec((8, 16), lambda i: (0, 0)),
                  pl.BlockSpec((1, 32), lambda i: (0, 0))],
        out_specs=pl.BlockSpec((BN, 32), lambda i: (i, 0)),
        out_shape=jax.ShapeDtypeStruct((N, 32), F32),
    )(acc2_pair, acc2_pair, table2, c2b, b2row)


def _make_sc_edge(DT, H, CH):
    p_col = H * CH
    h_vecs = p_col // 16
    mesh = plsc.VectorSubcoreMesh(core_axis_name="c", subcore_axis_name="s")

    @functools.partial(
        pl.kernel,
        mesh=mesh,
        compiler_params=pltpu.CompilerParams(use_tc_tiling_on_sc=False,
                                             needs_layout_passes=False),
        out_type=jax.ShapeDtypeStruct((NC * N, DT), F32),
        scratch_types=[
            pltpu.VMEM_SHARED((N, DT), F32),
            pltpu.VMEM((NCHUNK, CHUNK), jnp.int32),
            pltpu.VMEM((2, CHUNK, DT), F32),
            pltpu.VMEM((2, CHUNK, 16), F32),
            pltpu.VMEM((2, CHUNK), jnp.int32),
            pltpu.VMEM((2, CHUNK), jnp.int32),
            pltpu.VMEM((16,), F32),
            pltpu.SemaphoreType.DMA,
            pltpu.SemaphoreType.DMA,
        ],
    )
    def sc_edge(table_hbm, sdst_hbm, c_hbm, pk_hbm, zeros_hbm,
                out_hbm, acc_sh, pk_v, rows_v, sdr_v, usrc_v, udst_v, c_v,
                gsem0, gsem1):
        cid = lax.axis_index("c")
        sid = lax.axis_index("s")
        wid = sid * NC + cid
        pltpu.sync_copy(pk_hbm.at[wid], pk_v)
        pltpu.sync_copy(c_hbm, c_v)
        pltpu.sync_copy(zeros_hbm.at[pl.ds(sid * RPS, RPS)],
                        acc_sh.at[pl.ds(sid * RPS, RPS)])
        plsc.subcore_barrier()

        cvec = c_v[...]
        lane = lax.iota(jnp.int32, 16)
        gsems = (gsem0, gsem1)

        def unpack(j, b):
            def up(g, carry):
                w = pk_v[j, pl.ds(g * 16, 16)]
                usrc_v[b, pl.ds(g * 16, 16)] = w & 0xFFFF
                udst_v[b, pl.ds(g * 16, 16)] = lax.shift_right_logical(w, 16)
                return carry
            lax.fori_loop(0, CHUNK // 16, up, 0)

        def issue_gather(b):
            pltpu.async_copy(table_hbm.at[usrc_v.at[b]], rows_v.at[b],
                             gsems[b])
            pltpu.async_copy(sdst_hbm.at[udst_v.at[b]], sdr_v.at[b],
                             gsems[b])

        def wait_gather(b):
            pltpu.make_async_copy(table_hbm.at[usrc_v.at[b]], rows_v.at[b],
                                  gsems[b]).wait()
            pltpu.make_async_copy(sdst_hbm.at[udst_v.at[b]], sdr_v.at[b],
                                  gsems[b]).wait()

        def make_edge_body(b):
            def edge_body(e, carry):
                sv = rows_v[b, e, pl.ds(p_col, 16)]
                dv = sdr_v[b, e, pl.ds(0, 16)]
                p = jnp.exp(_leaky(sv + dv) - cvec)
                p = jnp.where(lane < H, p, 0.0)
                rows_v[b, e, pl.ds(p_col, 16)] = p
                for v in range(h_vecs):
                    ps = jnp.broadcast_to(p[(v * 16) // CH], (16,))
                    rows_v[b, e, pl.ds(v * 16, 16)] = (
                        rows_v[b, e, pl.ds(v * 16, 16)] * ps)
                return carry
            return edge_body

        unpack(0, 0)
        issue_gather(0)

        def pair_body(j2, carry):
            for b in (0, 1):
                j = 2 * j2 + b
                bn = 1 - b

                @pl.when(j < NCHUNK)
                def _slot():
                    @pl.when(j + 1 < NCHUNK)
                    def _prefetch():
                        unpack(j + 1, bn)
                        issue_gather(bn)
                    wait_gather(b)
                    lax.fori_loop(0, CHUNK, make_edge_body(b), 0)
                    pltpu.sync_copy(rows_v.at[b], acc_sh.at[udst_v.at[b]],
                                    add=True)
            return carry

        lax.fori_loop(0, (NCHUNK + 1) // 2, pair_body, 0)
        plsc.subcore_barrier()
        pltpu.sync_copy(acc_sh.at[pl.ds(sid * RPS, RPS)],
                        out_hbm.at[pl.ds(cid * N + sid * RPS, RPS)])

    return sc_edge


_sc_edge1 = _make_sc_edge(144, 8, 16)
_sc_edge2 = _make_sc_edge(48, 1, 32)


def kernel(x, edge_index, W1, att_src1, att_dst1, b1,
           W2, att_src2, att_dst2, b2):
    src = edge_index[0].astype(jnp.int32).reshape(NW, NCHUNK, CHUNK)
    dst = edge_index[1].astype(jnp.int32).reshape(NW, NCHUNK, CHUNK)
    pk = src | (dst << 16)

    heads = jnp.arange(8, dtype=jnp.int32)
    cols = jnp.arange(128, dtype=jnp.int32)
    mask1 = (cols[:, None] // 16 == heads[None, :]).astype(F32)
    a1s = att_src1.reshape(-1)
    a1d = att_dst1.reshape(-1)
    A1 = jnp.concatenate([a1s[:, None] * mask1, a1d[:, None] * mask1], 1)
    R1 = mask1.T
    a2s = att_src2.reshape(-1)
    a2d = att_dst2.reshape(-1)
    A2 = jnp.concatenate([a2s[:, None], a2d[:, None],
                          jnp.zeros((32, 14), F32)], 1)

    zeros144 = jnp.zeros((N, 144), F32)
    zeros48 = jnp.zeros((N, 48), F32)

    table1, ssd1, sdst1, mx1 = _pre1(x, W1, A1)
    m1 = jnp.max(mx1, axis=0)
    c1 = jnp.concatenate([m1[0:8] + m1[8:16], jnp.zeros((8,), F32)])
    acc1 = _sc_edge1(table1, sdst1, c1, pk, zeros144)
    c1b = jnp.broadcast_to(c1[None, :], (8, 16))

    table2, sdst2, mx2 = _mid(acc1, table1, ssd1, c1b, b1.reshape(1, 128),
                              W2, A2, R1)
    m2 = jnp.max(mx2, axis=0)
    c2 = jnp.concatenate([m2[0:1] + m2[1:2], jnp.zeros((15,), F32)])
    acc2 = _sc_edge2(table2, sdst2, c2, pk, zeros48)
    c2b = jnp.broadcast_to(c2[None, :], (8, 16))

    return _out(acc2, table2, c2b, b2.reshape(1, 32))


# speedup vs baseline: 91.1313x; 1.1507x over previous
import functools

import jax
import jax.numpy as jnp
from jax import lax
from jax.experimental import pallas as pl
from jax.experimental.pallas import tpu as pltpu
from jax.experimental.pallas import tpu_sc as plsc

F32 = jnp.float32
NEG_SLOPE = 0.2
N = 10000
E = 320000
NC, NS, LANES = 2, 16, 16
NW = NC * NS
EPW = E // NW
CHUNK = 80
NCHUNK = EPW // CHUNK
RPS = N // NS
BN = 400
GRID = N // BN


def _leaky(a):
    return jnp.where(a > 0, a, NEG_SLOPE * a)


def _pre1_body(x_ref, w_ref, a1_ref, table_ref, ssd_ref, sdst_ref, mx_ref):
    h = jnp.dot(x_ref[...], w_ref[...], preferred_element_type=F32)
    ssd = jnp.dot(h, a1_ref[...], preferred_element_type=F32)
    table_ref[...] = jnp.concatenate(
        [h, ssd[:, 0:8], jnp.zeros((BN, 8), F32)], axis=1)
    ssd_ref[...] = ssd
    sdst_ref[...] = jnp.concatenate(
        [ssd[:, 8:16], jnp.zeros((BN, 8), F32)], axis=1)
    mx_ref[...] = jnp.broadcast_to(jnp.max(ssd, axis=0, keepdims=True),
                                   (8, 16))


def _pre1(x, W1, A1):
    return pl.pallas_call(
        _pre1_body,
        grid=(GRID,),
        in_specs=[pl.BlockSpec((BN, 128), lambda i: (i, 0)),
                  pl.BlockSpec((128, 128), lambda i: (0, 0)),
                  pl.BlockSpec((128, 16), lambda i: (0, 0))],
        out_specs=[pl.BlockSpec((BN, 144), lambda i: (i, 0)),
                   pl.BlockSpec((BN, 16), lambda i: (i, 0)),
                   pl.BlockSpec((BN, 16), lambda i: (i, 0)),
                   pl.BlockSpec((8, 16), lambda i: (i, 0))],
        out_shape=[jax.ShapeDtypeStruct((N, 144), F32),
                   jax.ShapeDtypeStruct((N, 16), F32),
                   jax.ShapeDtypeStruct((N, 16), F32),
                   jax.ShapeDtypeStruct((8 * GRID, 16), F32)],
    )(x, W1, A1)


def _mid_body(acc0_ref, acc1_ref, table_ref, ssd_ref, c1_ref, b1_ref, w2_ref,
              a2_ref, r1_ref, table2_ref, sdst2_ref, mx2_ref):
    h1 = table_ref[:, 0:128]
    ssd = ssd_ref[...]
    p_self = jnp.exp(_leaky(ssd[:, 0:8] + ssd[:, 8:16]) - c1_ref[0:1, 0:8])
    acc = acc0_ref[...] + acc1_ref[...]
    num = acc[:, 0:128] + h1 * jnp.dot(p_self, r1_ref[...],
                                       preferred_element_type=F32)
    den = jnp.dot(acc[:, 128:136] + p_self, r1_ref[...],
                  preferred_element_type=F32) + 1e-16
    o1 = num / den + b1_ref[...]
    h1a = jnp.where(o1 > 0, o1, jnp.exp(jnp.minimum(o1, 0.0)) - 1.0)
    h2 = jnp.dot(h1a, w2_ref[...], preferred_element_type=F32)
    ssd2 = jnp.dot(h2, a2_ref[...], preferred_element_type=F32)
    table2_ref[...] = jnp.concatenate([h2, ssd2], axis=1)
    sdst2_ref[...] = jnp.concatenate(
        [ssd2[:, 1:2], jnp.zeros((BN, 15), F32)], axis=1)
    mx2_ref[...] = jnp.broadcast_to(jnp.max(ssd2, axis=0, keepdims=True),
                                    (8, 16))


def _mid(acc1_pair, table1, ssd1, c1b, b1row, W2, A2, R1):
    return pl.pallas_call(
        _mid_body,
        grid=(GRID,),
        in_specs=[pl.BlockSpec((BN, 144), lambda i: (i, 0)),
                  pl.BlockSpec((BN, 144), lambda i: (i + GRID, 0)),
                  pl.BlockSpec((BN, 144), lambda i: (i, 0)),
                  pl.BlockSpec((BN, 16), lambda i: (i, 0)),
                  pl.BlockSpec((8, 16), lambda i: (0, 0)),
                  pl.BlockSpec((1, 128), lambda i: (0, 0)),
                  pl.BlockSpec((128, 32), lambda i: (0, 0)),
                  pl.BlockSpec((32, 16), lambda i: (0, 0)),
                  pl.BlockSpec((8, 128), lambda i: (0, 0))],
        out_specs=[pl.BlockSpec((BN, 48), lambda i: (i, 0)),
                   pl.BlockSpec((BN, 16), lambda i: (i, 0)),
                   pl.BlockSpec((8, 16), lambda i: (i, 0))],
        out_shape=[jax.ShapeDtypeStruct((N, 48), F32),
                   jax.ShapeDtypeStruct((N, 16), F32),
                   jax.ShapeDtypeStruct((8 * GRID, 16), F32)],
    )(acc1_pair, acc1_pair, table1, ssd1, c1b, b1row, W2, A2, R1)


def _out_body(acc0_ref, acc1_ref, table2_ref, c2_ref, b2_ref, o_ref):
    h2 = table2_ref[:, 0:32]
    s2s = table2_ref[:, 32:33]
    s2d = table2_ref[:, 33:34]
    p_self = jnp.exp(_leaky(s2s + s2d) - c2_ref[0:1, 0:1])
    acc = acc0_ref[...] + acc1_ref[...]
    num = acc[:, 0:32] + h2 * p_self
    den = acc[:, 32:33] + p_self + 1e-16
    o2 = num / den + b2_ref[...]
    m = jnp.max(o2, axis=1, keepdims=True)
    z = o2 - m
    o_ref[...] = z - jnp.log(jnp.sum(jnp.exp(z), axis=1, keepdims=True))


def _out(acc2_pair, table2, c2b, b2row):
    return pl.pallas_call(
        _out_body,
        grid=(GRID,),
        in_specs=[pl.BlockSpec((BN, 48), lambda i: (i, 0)),
                  pl.BlockSpec((BN, 48), lambda i: (i + GRID, 0)),
                  pl.BlockSpec((BN, 48), lambda i: (i, 0)),
                  pl.BlockSpec((8, 16), lambda i: (0, 0)),
                  pl.BlockSpec((1, 32), lambda i: (0, 0))],
        out_specs=pl.BlockSpec((BN, 32), lambda i: (i, 0)),
        out_shape=jax.ShapeDtypeStruct((N, 32), F32),
    )(acc2_pair, acc2_pair, table2, c2b, b2row)


def _make_sc_edge(DT, H, CH):
    p_col = H * CH
    h_vecs = p_col // 16
    mesh = plsc.VectorSubcoreMesh(core_axis_name="c", subcore_axis_name="s")

    @functools.partial(
        pl.kernel,
        mesh=mesh,
        compiler_params=pltpu.CompilerParams(use_tc_tiling_on_sc=False,
                                             needs_layout_passes=False),
        out_type=jax.ShapeDtypeStruct((NC * N, DT), F32),
        scratch_types=[
            pltpu.VMEM_SHARED((N, DT), F32),
            pltpu.VMEM((3, 1, CHUNK), jnp.int32),
            pltpu.VMEM((3, CHUNK, DT), F32),
            pltpu.VMEM((3, CHUNK, 16), F32),
            pltpu.VMEM((3, CHUNK), jnp.int32),
            pltpu.VMEM((3, CHUNK), jnp.int32),
            pltpu.VMEM((16,), F32),
            [pltpu.SemaphoreType.DMA] * 3,
            [pltpu.SemaphoreType.DMA] * 3,
            [pltpu.SemaphoreType.DMA] * 3,
        ],
    )
    def sc_edge(table_hbm, sdst_hbm, c_hbm, pk_hbm, zeros_hbm,
                out_hbm, acc_sh, pk_v, rows_v, sdr_v, usrc_v, udst_v, c_v,
                psems, gsems, ssems):
        cid = lax.axis_index("c")
        sid = lax.axis_index("s")
        wid = sid * NC + cid
        pltpu.sync_copy(c_hbm, c_v)
        pltpu.sync_copy(pk_hbm.at[wid, pl.ds(0, 1)], pk_v.at[0])
        pltpu.sync_copy(zeros_hbm.at[pl.ds(sid * RPS, RPS)],
                        acc_sh.at[pl.ds(sid * RPS, RPS)])
        plsc.subcore_barrier()

        cvec = c_v[...]
        lane = lax.iota(jnp.int32, 16)

        def issue_pk(j, s):
            pltpu.async_copy(pk_hbm.at[wid, pl.ds(j, 1)], pk_v.at[s],
                             psems[s])

        def wait_pk(j, s):
            pltpu.make_async_copy(pk_hbm.at[wid, pl.ds(j, 1)], pk_v.at[s],
                                  psems[s]).wait()

        def unpack(s):
            def up(g, carry):
                w = pk_v[s, 0, pl.ds(g * 16, 16)]
                usrc_v[s, pl.ds(g * 16, 16)] = w & 0xFFFF
                udst_v[s, pl.ds(g * 16, 16)] = lax.shift_right_logical(w, 16)
                return carry
            lax.fori_loop(0, CHUNK // 16, up, 0)

        def issue_gather(s):
            pltpu.async_copy(table_hbm.at[usrc_v.at[s]], rows_v.at[s],
                             gsems[s])
            pltpu.async_copy(sdst_hbm.at[udst_v.at[s]], sdr_v.at[s],
                             gsems[s])

        def wait_gather(s):
            pltpu.make_async_copy(table_hbm.at[usrc_v.at[s]], rows_v.at[s],
                                  gsems[s]).wait()
            pltpu.make_async_copy(sdst_hbm.at[udst_v.at[s]], sdr_v.at[s],
                                  gsems[s]).wait()

        def issue_scatter(s):
            pltpu.async_copy(rows_v.at[s], acc_sh.at[udst_v.at[s]],
                             ssems[s], add=True)

        def wait_scatter(s):
            pltpu.make_async_copy(rows_v.at[s], acc_sh.at[udst_v.at[s]],
                                  ssems[s]).wait()

        def make_edge_body(s):
            def edge_body(e, carry):
                sv = rows_v[s, e, pl.ds(p_col, 16)]
                dv = sdr_v[s, e, pl.ds(0, 16)]
                p = jnp.exp(_leaky(sv + dv) - cvec)
                p = jnp.where(lane < H, p, 0.0)
                rows_v[s, e, pl.ds(p_col, 16)] = p
                for v in range(h_vecs):
                    ps = jnp.broadcast_to(p[(v * 16) // CH], (16,))
                    rows_v[s, e, pl.ds(v * 16, 16)] = (
                        rows_v[s, e, pl.ds(v * 16, 16)] * ps)
                return carry
            return edge_body

        unpack(0)
        issue_gather(0)
        issue_pk(1, 1)

        def trip_body(j3, carry):
            for k in (0, 1, 2):
                j = 3 * j3 + k
                kp = (k + 1) % 3

                @pl.when(j < NCHUNK)
                def _slot():
                    @pl.when(j + 2 < NCHUNK)
                    def _pkpre():
                        issue_pk(j + 2, (k + 2) % 3)

                    @pl.when(j + 1 < NCHUNK)
                    def _gpre():
                        @pl.when(j >= 2)
                        def _wscat():
                            wait_scatter(kp)
                        wait_pk(j + 1, kp)
                        unpack(kp)
                        issue_gather(kp)

                    wait_gather(k)
                    lax.fori_loop(0, CHUNK, make_edge_body(k), 0,
                                  unroll=2)
                    issue_scatter(k)
            return carry

        lax.fori_loop(0, (NCHUNK + 2) // 3, trip_body, 0)
        for jj in (NCHUNK - 3, NCHUNK - 2, NCHUNK - 1):
            wait_scatter(jj % 3)
        plsc.subcore_barrier()
        pltpu.sync_copy(acc_sh.at[pl.ds(sid * RPS, RPS)],
                        out_hbm.at[pl.ds(cid * N + sid * RPS, RPS)])

    return sc_edge


_sc_edge1 = _make_sc_edge(144, 8, 16)
_sc_edge2 = _make_sc_edge(48, 1, 32)


def kernel(x, edge_index, W1, att_src1, att_dst1, b1,
           W2, att_src2, att_dst2, b2):
    src = edge_index[0].astype(jnp.int32).reshape(NW, NCHUNK, CHUNK)
    dst = edge_index[1].astype(jnp.int32).reshape(NW, NCHUNK, CHUNK)
    pk = src | (dst << 16)

    heads = jnp.arange(8, dtype=jnp.int32)
    cols = jnp.arange(128, dtype=jnp.int32)
    mask1 = (cols[:, None] // 16 == heads[None, :]).astype(F32)
    a1s = att_src1.reshape(-1)
    a1d = att_dst1.reshape(-1)
    A1 = jnp.concatenate([a1s[:, None] * mask1, a1d[:, None] * mask1], 1)
    R1 = mask1.T
    a2s = att_src2.reshape(-1)
    a2d = att_dst2.reshape(-1)
    A2 = jnp.concatenate([a2s[:, None], a2d[:, None],
                          jnp.zeros((32, 14), F32)], 1)

    zeros144 = jnp.zeros((N, 144), F32)
    zeros48 = jnp.zeros((N, 48), F32)

    table1, ssd1, sdst1, mx1 = _pre1(x, W1, A1)
    m1 = jnp.max(mx1, axis=0)
    c1 = jnp.concatenate([m1[0:8] + m1[8:16], jnp.zeros((8,), F32)])
    acc1 = _sc_edge1(table1, sdst1, c1, pk, zeros144)
    c1b = jnp.broadcast_to(c1[None, :], (8, 16))

    table2, sdst2, mx2 = _mid(acc1, table1, ssd1, c1b, b1.reshape(1, 128),
                              W2, A2, R1)
    m2 = jnp.max(mx2, axis=0)
    c2 = jnp.concatenate([m2[0:1] + m2[1:2], jnp.zeros((15,), F32)])
    acc2 = _sc_edge2(table2, sdst2, c2, pk, zeros48)
    c2b = jnp.broadcast_to(c2[None, :], (8, 16))

    return _out(acc2, table2, c2b, b2.reshape(1, 32))


# speedup vs baseline: 119.3507x; 1.3097x over previous
import functools

import jax
import jax.numpy as jnp
from jax import lax
from jax.experimental import pallas as pl
from jax.experimental.pallas import tpu as pltpu
from jax.experimental.pallas import tpu_sc as plsc

F32 = jnp.float32
NEG_SLOPE = 0.2
N = 10000
E = 320000
NC, NS, LANES = 2, 16, 16
NW = NC * NS
EPW = E // NW
CHUNK = 80
NCHUNK = EPW // CHUNK
RPS = N // NS
BN = 400
GRID = N // BN


def _leaky(a):
    return jnp.where(a > 0, a, NEG_SLOPE * a)


def _pre1_body(x_ref, w_ref, a1_ref, table_ref, ssd_ref, sdst_ref, mx_ref):
    h = jnp.dot(x_ref[...], w_ref[...], preferred_element_type=F32)
    ssd = jnp.dot(h, a1_ref[...], preferred_element_type=F32)
    table_ref[...] = jnp.concatenate(
        [h, ssd[:, 0:8], jnp.zeros((BN, 8), F32)], axis=1)
    ssd_ref[...] = ssd
    sdst_ref[...] = jnp.concatenate(
        [ssd[:, 8:16], jnp.zeros((BN, 8), F32)], axis=1)
    mx_ref[...] = jnp.broadcast_to(jnp.max(ssd, axis=0, keepdims=True),
                                   (8, 16))


def _pre1(x, W1, A1):
    return pl.pallas_call(
        _pre1_body,
        grid=(GRID,),
        in_specs=[pl.BlockSpec((BN, 128), lambda i: (i, 0)),
                  pl.BlockSpec((128, 128), lambda i: (0, 0)),
                  pl.BlockSpec((128, 16), lambda i: (0, 0))],
        out_specs=[pl.BlockSpec((BN, 144), lambda i: (i, 0)),
                   pl.BlockSpec((BN, 16), lambda i: (i, 0)),
                   pl.BlockSpec((BN, 16), lambda i: (i, 0)),
                   pl.BlockSpec((8, 16), lambda i: (i, 0))],
        out_shape=[jax.ShapeDtypeStruct((N, 144), F32),
                   jax.ShapeDtypeStruct((N, 16), F32),
                   jax.ShapeDtypeStruct((N, 16), F32),
                   jax.ShapeDtypeStruct((8 * GRID, 16), F32)],
    )(x, W1, A1)


def _mid_body(acc0_ref, acc1_ref, table_ref, ssd_ref, c1_ref, b1_ref, w2_ref,
              a2_ref, r1_ref, table2_ref, sdst2_ref, mx2_ref):
    h1 = table_ref[:, 0:128]
    ssd = ssd_ref[...]
    p_self = jnp.exp(_leaky(ssd[:, 0:8] + ssd[:, 8:16]) - c1_ref[0:1, 0:8])
    acc = acc0_ref[...] + acc1_ref[...]
    num = acc[:, 0:128] + h1 * jnp.dot(p_self, r1_ref[...],
                                       preferred_element_type=F32)
    den = jnp.dot(acc[:, 128:136] + p_self, r1_ref[...],
                  preferred_element_type=F32) + 1e-16
    o1 = num / den + b1_ref[...]
    h1a = jnp.where(o1 > 0, o1, jnp.exp(jnp.minimum(o1, 0.0)) - 1.0)
    h2 = jnp.dot(h1a, w2_ref[...], preferred_element_type=F32)
    ssd2 = jnp.dot(h2, a2_ref[...], preferred_element_type=F32)
    table2_ref[...] = jnp.concatenate([h2, ssd2], axis=1)
    sdst2_ref[...] = jnp.concatenate(
        [ssd2[:, 1:2], jnp.zeros((BN, 15), F32)], axis=1)
    mx2_ref[...] = jnp.broadcast_to(jnp.max(ssd2, axis=0, keepdims=True),
                                    (8, 16))


def _mid(acc1_pair, table1, ssd1, c1b, b1row, W2, A2, R1):
    return pl.pallas_call(
        _mid_body,
        grid=(GRID,),
        in_specs=[pl.BlockSpec((BN, 144), lambda i: (i, 0)),
                  pl.BlockSpec((BN, 144), lambda i: (i + GRID, 0)),
                  pl.BlockSpec((BN, 144), lambda i: (i, 0)),
                  pl.BlockSpec((BN, 16), lambda i: (i, 0)),
                  pl.BlockSpec((8, 16), lambda i: (0, 0)),
                  pl.BlockSpec((1, 128), lambda i: (0, 0)),
                  pl.BlockSpec((128, 32), lambda i: (0, 0)),
                  pl.BlockSpec((32, 16), lambda i: (0, 0)),
                  pl.BlockSpec((8, 128), lambda i: (0, 0))],
        out_specs=[pl.BlockSpec((BN, 48), lambda i: (i, 0)),
                   pl.BlockSpec((BN, 16), lambda i: (i, 0)),
                   pl.BlockSpec((8, 16), lambda i: (i, 0))],
        out_shape=[jax.ShapeDtypeStruct((N, 48), F32),
                   jax.ShapeDtypeStruct((N, 16), F32),
                   jax.ShapeDtypeStruct((8 * GRID, 16), F32)],
    )(acc1_pair, acc1_pair, table1, ssd1, c1b, b1row, W2, A2, R1)


def _out_body(acc0_ref, acc1_ref, table2_ref, c2_ref, b2_ref, o_ref):
    h2 = table2_ref[:, 0:32]
    s2s = table2_ref[:, 32:33]
    s2d = table2_ref[:, 33:34]
    p_self = jnp.exp(_leaky(s2s + s2d) - c2_ref[0:1, 0:1])
    acc = acc0_ref[...] + acc1_ref[...]
    num = acc[:, 0:32] + h2 * p_self
    den = acc[:, 32:33] + p_self + 1e-16
    o2 = num / den + b2_ref[...]
    m = jnp.max(o2, axis=1, keepdims=True)
    z = o2 - m
    o_ref[...] = z - jnp.log(jnp.sum(jnp.exp(z), axis=1, keepdims=True))


def _out(acc2_pair, table2, c2b, b2row):
    return pl.pallas_call(
        _out_body,
        grid=(GRID,),
        in_specs=[pl.BlockSpec((BN, 48), lambda i: (i, 0)),
                  pl.BlockSpec((BN, 48), lambda i: (i + GRID, 0)),
                  pl.BlockSpec((BN, 48), lambda i: (i, 0)),
                  pl.BlockSpec((8, 16), lambda i: (0, 0)),
                  pl.BlockSpec((1, 32), lambda i: (0, 0))],
        out_specs=pl.BlockSpec((BN, 32), lambda i: (i, 0)),
        out_shape=jax.ShapeDtypeStruct((N, 32), F32),
    )(acc2_pair, acc2_pair, table2, c2b, b2row)


def _make_sc_edge(DT, H, CH):
    p_col = H * CH
    h_vecs = p_col // 16
    mesh = plsc.VectorSubcoreMesh(core_axis_name="c", subcore_axis_name="s")

    @functools.partial(
        pl.kernel,
        mesh=mesh,
        compiler_params=pltpu.CompilerParams(use_tc_tiling_on_sc=False,
                                             needs_layout_passes=False),
        out_type=jax.ShapeDtypeStruct((NC * N, DT), F32),
        scratch_types=[
            pltpu.VMEM_SHARED((N, DT), F32),
            pltpu.VMEM((3, 1, CHUNK), jnp.int32),
            pltpu.VMEM((3, CHUNK, DT), F32),
            pltpu.VMEM((3, CHUNK, 16), F32),
            pltpu.VMEM((3, CHUNK), jnp.int32),
            pltpu.VMEM((3, CHUNK), jnp.int32),
            pltpu.VMEM((16,), F32),
            [pltpu.SemaphoreType.DMA] * 3,
            [pltpu.SemaphoreType.DMA] * 3,
            [pltpu.SemaphoreType.DMA] * 3,
        ],
    )
    def sc_edge(table_hbm, sdst_hbm, c_hbm, pk_hbm, zeros_hbm,
                out_hbm, acc_sh, pk_v, rows_v, sdr_v, usrc_v, udst_v, c_v,
                psems, gsems, ssems):
        cid = lax.axis_index("c")
        sid = lax.axis_index("s")
        wid = sid * NC + cid
        pltpu.sync_copy(c_hbm, c_v)
        pltpu.sync_copy(pk_hbm.at[wid, pl.ds(0, 1)], pk_v.at[0])
        pltpu.sync_copy(zeros_hbm.at[pl.ds(sid * RPS, RPS)],
                        acc_sh.at[pl.ds(sid * RPS, RPS)])
        plsc.subcore_barrier()

        cvec = c_v[...]
        lane = lax.iota(jnp.int32, 16)

        def issue_pk(j, s):
            pltpu.async_copy(pk_hbm.at[wid, pl.ds(j, 1)], pk_v.at[s],
                             psems[s])

        def wait_pk(j, s):
            pltpu.make_async_copy(pk_hbm.at[wid, pl.ds(j, 1)], pk_v.at[s],
                                  psems[s]).wait()

        def unpack(s):
            def up(g, carry):
                w = pk_v[s, 0, pl.ds(g * 16, 16)]
                usrc_v[s, pl.ds(g * 16, 16)] = w & 0xFFFF
                udst_v[s, pl.ds(g * 16, 16)] = lax.shift_right_logical(w, 16)
                return carry
            lax.fori_loop(0, CHUNK // 16, up, 0)

        def issue_gather(s):
            pltpu.async_copy(table_hbm.at[usrc_v.at[s]], rows_v.at[s],
                             gsems[s])
            pltpu.async_copy(sdst_hbm.at[udst_v.at[s]], sdr_v.at[s],
                             gsems[s])

        def wait_gather(s):
            pltpu.make_async_copy(table_hbm.at[usrc_v.at[s]], rows_v.at[s],
                                  gsems[s]).wait()
            pltpu.make_async_copy(sdst_hbm.at[udst_v.at[s]], sdr_v.at[s],
                                  gsems[s]).wait()

        def issue_scatter(s):
            pltpu.async_copy(rows_v.at[s], acc_sh.at[udst_v.at[s]],
                             ssems[s], add=True)

        def wait_scatter(s):
            pltpu.make_async_copy(rows_v.at[s], acc_sh.at[udst_v.at[s]],
                                  ssems[s]).wait()

        def make_group_body(s):
            def group_body(g, carry):
                eids = lane + g * 16
                pvs = []
                for h in range(H):
                    colv = jnp.full((16,), p_col + h, jnp.int32)
                    ssv = plsc.load_gather(rows_v.at[s], [eids, colv])
                    sdv = plsc.load_gather(
                        sdr_v.at[s], [eids, jnp.full((16,), h, jnp.int32)])
                    a = ssv + sdv
                    a = jnp.maximum(a, NEG_SLOPE * a)
                    pv = jnp.exp(a - jnp.broadcast_to(cvec[h], (16,)))
                    plsc.store_scatter(rows_v.at[s], [eids, colv], pv)
                    pvs.append(pv)
                for e16 in range(16):
                    e = g * 16 + e16
                    for v in range(h_vecs):
                        ps = jnp.broadcast_to(pvs[(v * 16) // CH][e16], (16,))
                        rows_v[s, e, pl.ds(v * 16, 16)] = (
                            rows_v[s, e, pl.ds(v * 16, 16)] * ps)
                return carry
            return group_body

        unpack(0)
        issue_gather(0)
        issue_pk(1, 1)

        def trip_body(j3, carry):
            for k in (0, 1, 2):
                j = 3 * j3 + k
                kp = (k + 1) % 3

                @pl.when(j < NCHUNK)
                def _slot():
                    @pl.when(j + 2 < NCHUNK)
                    def _pkpre():
                        issue_pk(j + 2, (k + 2) % 3)

                    @pl.when(j + 1 < NCHUNK)
                    def _gpre():
                        @pl.when(j >= 2)
                        def _wscat():
                            wait_scatter(kp)
                        wait_pk(j + 1, kp)
                        unpack(kp)
                        issue_gather(kp)

                    wait_gather(k)
                    lax.fori_loop(0, CHUNK // 16, make_group_body(k), 0)
                    issue_scatter(k)
            return carry

        lax.fori_loop(0, (NCHUNK + 2) // 3, trip_body, 0)
        for jj in (NCHUNK - 3, NCHUNK - 2, NCHUNK - 1):
            wait_scatter(jj % 3)
        plsc.subcore_barrier()
        pltpu.sync_copy(acc_sh.at[pl.ds(sid * RPS, RPS)],
                        out_hbm.at[pl.ds(cid * N + sid * RPS, RPS)])

    return sc_edge


_sc_edge1 = _make_sc_edge(144, 8, 16)
_sc_edge2 = _make_sc_edge(48, 1, 32)


def kernel(x, edge_index, W1, att_src1, att_dst1, b1,
           W2, att_src2, att_dst2, b2):
    src = edge_index[0].astype(jnp.int32).reshape(NW, NCHUNK, CHUNK)
    dst = edge_index[1].astype(jnp.int32).reshape(NW, NCHUNK, CHUNK)
    pk = src | (dst << 16)

    heads = jnp.arange(8, dtype=jnp.int32)
    cols = jnp.arange(128, dtype=jnp.int32)
    mask1 = (cols[:, None] // 16 == heads[None, :]).astype(F32)
    a1s = att_src1.reshape(-1)
    a1d = att_dst1.reshape(-1)
    A1 = jnp.concatenate([a1s[:, None] * mask1, a1d[:, None] * mask1], 1)
    R1 = mask1.T
    a2s = att_src2.reshape(-1)
    a2d = att_dst2.reshape(-1)
    A2 = jnp.concatenate([a2s[:, None], a2d[:, None],
                          jnp.zeros((32, 14), F32)], 1)

    zeros144 = jnp.zeros((N, 144), F32)
    zeros48 = jnp.zeros((N, 48), F32)

    table1, ssd1, sdst1, mx1 = _pre1(x, W1, A1)
    m1 = jnp.max(mx1, axis=0)
    c1 = jnp.concatenate([m1[0:8] + m1[8:16], jnp.zeros((8,), F32)])
    acc1 = _sc_edge1(table1, sdst1, c1, pk, zeros144)
    c1b = jnp.broadcast_to(c1[None, :], (8, 16))

    table2, sdst2, mx2 = _mid(acc1, table1, ssd1, c1b, b1.reshape(1, 128),
                              W2, A2, R1)
    m2 = jnp.max(mx2, axis=0)
    c2 = jnp.concatenate([m2[0:1] + m2[1:2], jnp.zeros((15,), F32)])
    acc2 = _sc_edge2(table2, sdst2, c2, pk, zeros48)
    c2b = jnp.broadcast_to(c2[None, :], (8, 16))

    return _out(acc2, table2, c2b, b2.reshape(1, 32))


# speedup vs baseline: 122.0371x; 1.0225x over previous
import functools

import jax
import jax.numpy as jnp
from jax import lax
from jax.experimental import pallas as pl
from jax.experimental.pallas import tpu as pltpu
from jax.experimental.pallas import tpu_sc as plsc

F32 = jnp.float32
NEG_SLOPE = 0.2
N = 10000
E = 320000
NC, NS, LANES = 2, 16, 16
NW = NC * NS
EPW = E // NW
CHUNK = 80
NCHUNK = EPW // CHUNK
RPS = N // NS
BN = 400
GRID = N // BN


def _leaky(a):
    return jnp.where(a > 0, a, NEG_SLOPE * a)


def _pre1_body(x_ref, w_ref, a1_ref, table_ref, ssd_ref, sdst_ref, mx_ref):
    h = jnp.dot(x_ref[...], w_ref[...], preferred_element_type=F32)
    ssd = jnp.dot(h, a1_ref[...], preferred_element_type=F32)
    table_ref[...] = jnp.concatenate(
        [h, ssd[:, 0:8], jnp.zeros((BN, 8), F32)], axis=1)
    ssd_ref[...] = ssd
    sdst_ref[...] = jnp.concatenate(
        [ssd[:, 8:16], jnp.zeros((BN, 8), F32)], axis=1)
    mx_ref[...] = jnp.broadcast_to(jnp.max(ssd, axis=0, keepdims=True),
                                   (8, 16))


def _pre1(x, W1, A1):
    return pl.pallas_call(
        _pre1_body,
        grid=(GRID,),
        in_specs=[pl.BlockSpec((BN, 128), lambda i: (i, 0)),
                  pl.BlockSpec((128, 128), lambda i: (0, 0)),
                  pl.BlockSpec((128, 16), lambda i: (0, 0))],
        out_specs=[pl.BlockSpec((BN, 144), lambda i: (i, 0)),
                   pl.BlockSpec((BN, 16), lambda i: (i, 0)),
                   pl.BlockSpec((BN, 16), lambda i: (i, 0)),
                   pl.BlockSpec((8, 16), lambda i: (i, 0))],
        out_shape=[jax.ShapeDtypeStruct((N, 144), F32),
                   jax.ShapeDtypeStruct((N, 16), F32),
                   jax.ShapeDtypeStruct((N, 16), F32),
                   jax.ShapeDtypeStruct((8 * GRID, 16), F32)],
    )(x, W1, A1)


def _mid_body(acc0_ref, acc1_ref, table_ref, ssd_ref, c1_ref, b1_ref, w2_ref,
              a2_ref, r1_ref, table2_ref, sdst2_ref, mx2_ref):
    h1 = table_ref[:, 0:128]
    ssd = ssd_ref[...]
    p_self = jnp.exp(_leaky(ssd[:, 0:8] + ssd[:, 8:16]) - c1_ref[0:1, 0:8])
    acc = acc0_ref[...] + acc1_ref[...]
    num = acc[:, 0:128] + h1 * jnp.dot(p_self, r1_ref[...],
                                       preferred_element_type=F32)
    den = jnp.dot(acc[:, 128:136] + p_self, r1_ref[...],
                  preferred_element_type=F32) + 1e-16
    o1 = num / den + b1_ref[...]
    h1a = jnp.where(o1 > 0, o1, jnp.exp(jnp.minimum(o1, 0.0)) - 1.0)
    h2 = jnp.dot(h1a, w2_ref[...], preferred_element_type=F32)
    ssd2 = jnp.dot(h2, a2_ref[...], preferred_element_type=F32)
    table2_ref[...] = jnp.concatenate([h2, ssd2], axis=1)
    sdst2_ref[...] = jnp.concatenate(
        [ssd2[:, 1:2], jnp.zeros((BN, 15), F32)], axis=1)
    mx2_ref[...] = jnp.broadcast_to(jnp.max(ssd2, axis=0, keepdims=True),
                                    (8, 16))


def _mid(acc1_pair, table1, ssd1, c1b, b1row, W2, A2, R1):
    return pl.pallas_call(
        _mid_body,
        grid=(GRID,),
        in_specs=[pl.BlockSpec((BN, 144), lambda i: (i, 0)),
                  pl.BlockSpec((BN, 144), lambda i: (i + GRID, 0)),
                  pl.BlockSpec((BN, 144), lambda i: (i, 0)),
                  pl.BlockSpec((BN, 16), lambda i: (i, 0)),
                  pl.BlockSpec((8, 16), lambda i: (0, 0)),
                  pl.BlockSpec((1, 128), lambda i: (0, 0)),
                  pl.BlockSpec((128, 32), lambda i: (0, 0)),
                  pl.BlockSpec((32, 16), lambda i: (0, 0)),
                  pl.BlockSpec((8, 128), lambda i: (0, 0))],
        out_specs=[pl.BlockSpec((BN, 48), lambda i: (i, 0)),
                   pl.BlockSpec((BN, 16), lambda i: (i, 0)),
                   pl.BlockSpec((8, 16), lambda i: (i, 0))],
        out_shape=[jax.ShapeDtypeStruct((N, 48), F32),
                   jax.ShapeDtypeStruct((N, 16), F32),
                   jax.ShapeDtypeStruct((8 * GRID, 16), F32)],
    )(acc1_pair, acc1_pair, table1, ssd1, c1b, b1row, W2, A2, R1)


def _out_body(acc0_ref, acc1_ref, table2_ref, c2_ref, b2_ref, o_ref):
    h2 = table2_ref[:, 0:32]
    s2s = table2_ref[:, 32:33]
    s2d = table2_ref[:, 33:34]
    p_self = jnp.exp(_leaky(s2s + s2d) - c2_ref[0:1, 0:1])
    acc = acc0_ref[...] + acc1_ref[...]
    num = acc[:, 0:32] + h2 * p_self
    den = acc[:, 32:33] + p_self + 1e-16
    o2 = num / den + b2_ref[...]
    m = jnp.max(o2, axis=1, keepdims=True)
    z = o2 - m
    o_ref[...] = z - jnp.log(jnp.sum(jnp.exp(z), axis=1, keepdims=True))


def _out(acc2_pair, table2, c2b, b2row):
    return pl.pallas_call(
        _out_body,
        grid=(GRID,),
        in_specs=[pl.BlockSpec((BN, 48), lambda i: (i, 0)),
                  pl.BlockSpec((BN, 48), lambda i: (i + GRID, 0)),
                  pl.BlockSpec((BN, 48), lambda i: (i, 0)),
                  pl.BlockSpec((8, 16), lambda i: (0, 0)),
                  pl.BlockSpec((1, 32), lambda i: (0, 0))],
        out_specs=pl.BlockSpec((BN, 32), lambda i: (i, 0)),
        out_shape=jax.ShapeDtypeStruct((N, 32), F32),
    )(acc2_pair, acc2_pair, table2, c2b, b2row)


def _make_sc_edge(DT, H, CH):
    p_col = H * CH
    h_vecs = p_col // 16
    mesh = plsc.VectorSubcoreMesh(core_axis_name="c", subcore_axis_name="s")

    @functools.partial(
        pl.kernel,
        mesh=mesh,
        compiler_params=pltpu.CompilerParams(use_tc_tiling_on_sc=False,
                                             needs_layout_passes=False),
        out_type=jax.ShapeDtypeStruct((NC * N, DT), F32),
        scratch_types=[
            pltpu.VMEM_SHARED((N, DT), F32),
            pltpu.VMEM((3, 1, CHUNK), jnp.int32),
            pltpu.VMEM((3, CHUNK, DT), F32),
            pltpu.VMEM((3, CHUNK, 16), F32),
            pltpu.VMEM((3, CHUNK), jnp.int32),
            pltpu.VMEM((3, CHUNK), jnp.int32),
            pltpu.VMEM((16,), F32),
            [pltpu.SemaphoreType.DMA] * 3,
            [pltpu.SemaphoreType.DMA] * 3,
            [pltpu.SemaphoreType.DMA] * 3,
        ],
    )
    def sc_edge(table_hbm, sdst_hbm, c_hbm, pk_hbm,
                out_hbm, acc_sh, pk_v, rows_v, sdr_v, usrc_v, udst_v, c_v,
                psems, gsems, ssems):
        cid = lax.axis_index("c")
        sid = lax.axis_index("s")
        wid = sid * NC + cid
        pltpu.sync_copy(c_hbm, c_v)
        pltpu.sync_copy(pk_hbm.at[wid, pl.ds(0, 1)], pk_v.at[0])

        def zr(i, carry):
            for t in range(DT // 16):
                rows_v[0, i, pl.ds(t * 16, 16)] = jnp.zeros((16,), F32)
            return carry
        lax.fori_loop(0, CHUNK, zr, 0)
        zbase = sid * RPS
        for i in range(RPS // CHUNK):
            pltpu.sync_copy(rows_v.at[0],
                            acc_sh.at[pl.ds(zbase + i * CHUNK, CHUNK)])
        zrem = RPS % CHUNK
        if zrem:
            pltpu.sync_copy(rows_v.at[0, pl.ds(0, zrem)],
                            acc_sh.at[pl.ds(zbase + RPS - zrem, zrem)])
        plsc.subcore_barrier()

        cvec = c_v[...]
        lane = lax.iota(jnp.int32, 16)

        def issue_pk(j, s):
            pltpu.async_copy(pk_hbm.at[wid, pl.ds(j, 1)], pk_v.at[s],
                             psems[s])

        def wait_pk(j, s):
            pltpu.make_async_copy(pk_hbm.at[wid, pl.ds(j, 1)], pk_v.at[s],
                                  psems[s]).wait()

        def unpack(s):
            def up(g, carry):
                w = pk_v[s, 0, pl.ds(g * 16, 16)]
                usrc_v[s, pl.ds(g * 16, 16)] = w & 0xFFFF
                udst_v[s, pl.ds(g * 16, 16)] = lax.shift_right_logical(w, 16)
                return carry
            lax.fori_loop(0, CHUNK // 16, up, 0)

        def issue_gather(s):
            pltpu.async_copy(table_hbm.at[usrc_v.at[s]], rows_v.at[s],
                             gsems[s])
            pltpu.async_copy(sdst_hbm.at[udst_v.at[s]], sdr_v.at[s],
                             gsems[s])

        def wait_gather(s):
            pltpu.make_async_copy(table_hbm.at[usrc_v.at[s]], rows_v.at[s],
                                  gsems[s]).wait()
            pltpu.make_async_copy(sdst_hbm.at[udst_v.at[s]], sdr_v.at[s],
                                  gsems[s]).wait()

        def issue_scatter(s):
            pltpu.async_copy(rows_v.at[s], acc_sh.at[udst_v.at[s]],
                             ssems[s], add=True)

        def wait_scatter(s):
            pltpu.make_async_copy(rows_v.at[s], acc_sh.at[udst_v.at[s]],
                                  ssems[s]).wait()

        def make_group_body(s):
            def group_body(g, carry):
                eids = lane + g * 16
                pvs = []
                for h in range(H):
                    colv = jnp.full((16,), p_col + h, jnp.int32)
                    ssv = plsc.load_gather(rows_v.at[s], [eids, colv])
                    sdv = plsc.load_gather(
                        sdr_v.at[s], [eids, jnp.full((16,), h, jnp.int32)])
                    a = ssv + sdv
                    a = jnp.maximum(a, NEG_SLOPE * a)
                    pv = jnp.exp(a - jnp.broadcast_to(cvec[h], (16,)))
                    plsc.store_scatter(rows_v.at[s], [eids, colv], pv)
                    pvs.append(pv)
                for e16 in range(16):
                    e = g * 16 + e16
                    for v in range(h_vecs):
                        ps = jnp.broadcast_to(pvs[(v * 16) // CH][e16], (16,))
                        rows_v[s, e, pl.ds(v * 16, 16)] = (
                            rows_v[s, e, pl.ds(v * 16, 16)] * ps)
                return carry
            return group_body

        unpack(0)
        issue_gather(0)
        issue_pk(1, 1)

        def trip_body(j3, carry):
            for k in (0, 1, 2):
                j = 3 * j3 + k
                kp = (k + 1) % 3

                @pl.when(j < NCHUNK)
                def _slot():
                    @pl.when(j + 2 < NCHUNK)
                    def _pkpre():
                        issue_pk(j + 2, (k + 2) % 3)

                    @pl.when(j + 1 < NCHUNK)
                    def _gpre():
                        @pl.when(j >= 2)
                        def _wscat():
                            wait_scatter(kp)
                        wait_pk(j + 1, kp)
                        unpack(kp)
                        issue_gather(kp)

                    wait_gather(k)
                    lax.fori_loop(0, CHUNK // 16, make_group_body(k), 0)
                    issue_scatter(k)
            return carry

        lax.fori_loop(0, (NCHUNK + 2) // 3, trip_body, 0)
        for jj in (NCHUNK - 3, NCHUNK - 2, NCHUNK - 1):
            wait_scatter(jj % 3)
        plsc.subcore_barrier()
        pltpu.sync_copy(acc_sh.at[pl.ds(sid * RPS, RPS)],
                        out_hbm.at[pl.ds(cid * N + sid * RPS, RPS)])

    return sc_edge


_sc_edge1 = _make_sc_edge(144, 8, 16)
_sc_edge2 = _make_sc_edge(48, 1, 32)


def kernel(x, edge_index, W1, att_src1, att_dst1, b1,
           W2, att_src2, att_dst2, b2):
    src = edge_index[0].astype(jnp.int32).reshape(NW, NCHUNK, CHUNK)
    dst = edge_index[1].astype(jnp.int32).reshape(NW, NCHUNK, CHUNK)
    pk = src | (dst << 16)

    heads = jnp.arange(8, dtype=jnp.int32)
    cols = jnp.arange(128, dtype=jnp.int32)
    mask1 = (cols[:, None] // 16 == heads[None, :]).astype(F32)
    a1s = att_src1.reshape(-1)
    a1d = att_dst1.reshape(-1)
    A1 = jnp.concatenate([a1s[:, None] * mask1, a1d[:, None] * mask1], 1)
    R1 = mask1.T
    a2s = att_src2.reshape(-1)
    a2d = att_dst2.reshape(-1)
    A2 = jnp.concatenate([a2s[:, None], a2d[:, None],
                          jnp.zeros((32, 14), F32)], 1)

    table1, ssd1, sdst1, mx1 = _pre1(x, W1, A1)
    m1 = jnp.max(mx1, axis=0)
    c1 = jnp.concatenate([m1[0:8] + m1[8:16], jnp.zeros((8,), F32)])
    acc1 = _sc_edge1(table1, sdst1, c1, pk)
    c1b = jnp.broadcast_to(c1[None, :], (8, 16))

    table2, sdst2, mx2 = _mid(acc1, table1, ssd1, c1b, b1.reshape(1, 128),
                              W2, A2, R1)
    m2 = jnp.max(mx2, axis=0)
    c2 = jnp.concatenate([m2[0:1] + m2[1:2], jnp.zeros((15,), F32)])
    acc2 = _sc_edge2(table2, sdst2, c2, pk)
    c2b = jnp.broadcast_to(c2[None, :], (8, 16))

    return _out(acc2, table2, c2b, b2.reshape(1, 32))
